# baseline (speedup 1.0000x reference)
import functools

import jax
import jax.numpy as jnp
from jax import lax
from jax.experimental import pallas as pl
from jax.experimental.pallas import tpu as pltpu
from jax.experimental.pallas import tpu_sc as plsc

NUM_ENTITIES = 1000000
NUM_RELATIONS = 1000
EMBED_DIM = 64
BATCH = 16384

_INFO = plsc.get_sparse_core_info()
_NC = _INFO.num_cores
_NS = _INFO.num_subcores
_L = _INFO.num_lanes
_NW = _NC * _NS
_BPW = BATCH // _NW
_CHUNK = 128
_NCHUNK = _BPW // _CHUNK
_GROUPS = _BPW // _L


def _sqrt16(x):
    xc = jnp.maximum(x, jnp.float32(1e-35))
    i = plsc.bitcast(xc, jnp.int32)
    y = plsc.bitcast(jnp.int32(0x5F3759DF) - (i >> 1), jnp.float32)
    half = jnp.float32(0.5) * xc
    for _ in range(4):
        y = y * (jnp.float32(1.5) - half * y * y)
    return x * y


def _transe_body(head_hbm, rel_hbm, tail_hbm, ent_hbm, relemb_hbm, out_hbm,
                 hidx, ridx, tidx, hrows, rrows, trows, score_v, sem):
    wid = lax.axis_index("s") * _NC + lax.axis_index("c")
    base = wid * _BPW

    for j in range(_NCHUNK):
        off = base + j * _CHUNK
        pltpu.sync_copy(head_hbm.at[pl.ds(off, _CHUNK)], hidx.at[j])
        pltpu.sync_copy(rel_hbm.at[pl.ds(off, _CHUNK)], ridx.at[j])
        pltpu.sync_copy(tail_hbm.at[pl.ds(off, _CHUNK)], tidx.at[j])

    copies = []
    for j in range(_NCHUNK):
        rows = pl.ds(j * _CHUNK, _CHUNK)
        copies.append(pltpu.async_copy(ent_hbm.at[hidx.at[j]], hrows.at[rows], sem))
        copies.append(pltpu.async_copy(relemb_hbm.at[ridx.at[j]], rrows.at[rows], sem))
        copies.append(pltpu.async_copy(ent_hbm.at[tidx.at[j]], trows.at[rows], sem))
    for c in copies:
        c.wait()

    lane = lax.iota(jnp.int32, _L)
    zero = jnp.zeros((_L,), jnp.float32)

    def group_body(g, _):
        out_vec = zero
        for l in range(_L):
            row = g * _L + l
            acc = zero
            for c in range(EMBED_DIM // _L):
                cols = pl.ds(c * _L, _L)
                h = hrows[row, cols]
                r = rrows[row, cols]
                t = trows[row, cols]
                d = h + r - t
                acc = acc + d * d
            s = jnp.sum(acc)
            out_vec = jnp.where(lane == l, s, out_vec)
        score_v[pl.ds(g * _L, _L)] = _sqrt16(out_vec)
        return 0

    lax.fori_loop(0, _GROUPS, group_body, 0)

    pltpu.sync_copy(score_v, out_hbm.at[pl.ds(base, _BPW)])


@jax.jit
def kernel(head, relation, tail, entity_emb, relation_emb):
    mesh = plsc.VectorSubcoreMesh(core_axis_name="c", subcore_axis_name="s")
    k = functools.partial(
        pl.kernel,
        mesh=mesh,
        out_type=jax.ShapeDtypeStruct((BATCH,), jnp.float32),
        scratch_types=[
            pltpu.VMEM((_NCHUNK, _CHUNK), jnp.int32),
            pltpu.VMEM((_NCHUNK, _CHUNK), jnp.int32),
            pltpu.VMEM((_NCHUNK, _CHUNK), jnp.int32),
            pltpu.VMEM((_BPW, EMBED_DIM), jnp.float32),
            pltpu.VMEM((_BPW, EMBED_DIM), jnp.float32),
            pltpu.VMEM((_BPW, EMBED_DIM), jnp.float32),
            pltpu.VMEM((_BPW,), jnp.float32),
            pltpu.SemaphoreType.DMA,
        ],
        compiler_params=pltpu.CompilerParams(
            needs_layout_passes=False, use_tc_tiling_on_sc=False),
    )(_transe_body)
    return k(head, relation, tail, entity_emb, relation_emb)
